# baseline (device time: 63618 ns/iter reference)
import jax
import jax.numpy as jnp
from jax import lax
from jax.experimental import pallas as pl
from jax.experimental.pallas import tpu as pltpu

N_DEV = 8
BLK = 64


def kernel(x, Wq, K_ext, V_ext, Wo):
    B, Sq, Dm = x.shape
    _, Skv, _, Dh = K_ext.shape
    H_loc = Wq.shape[1] // Dh
    BH = B * H_loc
    R = B * Sq

    my = lax.axis_index("i")

    xb = x.reshape(R, Dm).astype(jnp.bfloat16)
    wq = Wq.astype(jnp.bfloat16)
    wo = Wo.astype(jnp.bfloat16)
    k_loc = lax.dynamic_slice_in_dim(K_ext, my * H_loc, H_loc, axis=2)
    v_loc = lax.dynamic_slice_in_dim(V_ext, my * H_loc, H_loc, axis=2)
    k_loc = k_loc.transpose(0, 2, 1, 3).reshape(BH, Skv, Dh).astype(jnp.bfloat16)
    v_loc = v_loc.transpose(0, 2, 1, 3).reshape(BH, Skv, Dh).astype(jnp.bfloat16)

    def body(x_ref, wq_ref, k_ref, v_ref, wo_ref, out_ref,
             comm_ref, ctx_ref, send_sems, recv_sems):
        my_pos = lax.axis_index("i")
        left = (my_pos - 1) % N_DEV
        right = (my_pos + 1) % N_DEV

        barrier_sem = pltpu.get_barrier_semaphore()
        for nbr in (left, right):
            pl.semaphore_signal(
                barrier_sem, inc=1,
                device_id=(nbr,), device_id_type=pl.DeviceIdType.MESH,
            )
        pl.semaphore_wait(barrier_sem, 2)

        q = jnp.dot(x_ref[...], wq_ref[...],
                    preferred_element_type=jnp.float32)
        rows = lax.broadcasted_iota(jnp.int32, (Sq, Skv), 0) // BLK
        cols = lax.broadcasted_iota(jnp.int32, (Sq, Skv), 1) // BLK
        mask = cols <= rows

        for b in range(B):
            for h in range(H_loc):
                bh = b * H_loc + h
                qbh = q[b * Sq:(b + 1) * Sq, h * Dh:(h + 1) * Dh]
                s = lax.dot_general(
                    qbh.astype(jnp.bfloat16), k_ref[bh],
                    (((1,), (1,)), ((), ())),
                    preferred_element_type=jnp.float32,
                ) * 0.125
                s = jnp.where(mask, s, -1e9)
                m = jnp.max(s, axis=-1, keepdims=True)
                w = jnp.exp(s - m)
                w = w / jnp.sum(w, axis=-1, keepdims=True)
                ctx = jnp.dot(w.astype(jnp.bfloat16), v_ref[bh],
                              preferred_element_type=jnp.float32)
                ctx_ref[b * Sq:(b + 1) * Sq, h * Dh:(h + 1) * Dh] = (
                    ctx.astype(jnp.bfloat16))

        partial = jnp.dot(ctx_ref[...], wo_ref[...],
                          preferred_element_type=jnp.float32)
        out_ref[...] = partial
        comm_ref[0] = partial.astype(jnp.bfloat16)

        for hop in range(N_DEV - 1):
            rdma = pltpu.make_async_remote_copy(
                src_ref=comm_ref.at[hop],
                dst_ref=comm_ref.at[hop + 1],
                send_sem=send_sems.at[hop],
                recv_sem=recv_sems.at[hop],
                device_id=(right,),
                device_id_type=pl.DeviceIdType.MESH,
            )
            rdma.start()
            rdma.wait()
            out_ref[...] = out_ref[...] + comm_ref[hop + 1].astype(jnp.float32)

    out = pl.pallas_call(
        body,
        out_shape=jax.ShapeDtypeStruct((R, Dm), jnp.float32),
        in_specs=[pl.BlockSpec(memory_space=pltpu.VMEM)] * 5,
        out_specs=pl.BlockSpec(memory_space=pltpu.VMEM),
        scratch_shapes=[
            pltpu.VMEM((N_DEV, R, Dm), jnp.bfloat16),
            pltpu.VMEM((R, H_loc * Dh), jnp.bfloat16),
            pltpu.SemaphoreType.DMA((N_DEV - 1,)),
            pltpu.SemaphoreType.DMA((N_DEV - 1,)),
        ],
        compiler_params=pltpu.CompilerParams(collective_id=0),
    )(xb, wq, k_loc, v_loc, wo)

    return out.reshape(B, Sq, Dm)


# device time: 32474 ns/iter; 1.9590x vs baseline; 1.9590x over previous
import jax
import jax.numpy as jnp
from jax import lax
from jax.experimental import pallas as pl
from jax.experimental.pallas import tpu as pltpu

N_DEV = 8
BLK = 64


def kernel(x, Wq, K_ext, V_ext, Wo):
    B, Sq, Dm = x.shape
    _, Skv, _, Dh = K_ext.shape
    H_loc = Wq.shape[1] // Dh
    BH = B * H_loc
    R = B * Sq
    R_BLK = R // N_DEV

    my = lax.axis_index("i")

    xb = x.reshape(R, Dm).astype(jnp.bfloat16)
    wq = Wq.astype(jnp.bfloat16)
    wo = Wo.astype(jnp.bfloat16)
    k_loc = lax.dynamic_slice_in_dim(K_ext, my * H_loc, H_loc, axis=2)
    v_loc = lax.dynamic_slice_in_dim(V_ext, my * H_loc, H_loc, axis=2)
    k_loc = k_loc.transpose(0, 2, 1, 3).reshape(BH, Skv, Dh).astype(jnp.bfloat16)
    v_loc = v_loc.transpose(0, 2, 1, 3).reshape(BH, Skv, Dh).astype(jnp.bfloat16)

    RS_MASKS = (4, 2, 1)
    RS_OFF = {4: 0, 2: 4 * R_BLK, 1: 6 * R_BLK}
    AG_MASKS = (1, 2, 4)

    def body(x_ref, wq_ref, k_ref, v_ref, wo_ref, out_ref,
             st_ref, rs_ref, ag_ref, ctx_ref, send_sems, recv_sems):
        my_pos = lax.axis_index("i")

        barrier_sem = pltpu.get_barrier_semaphore()
        for m in (1, 2, 4):
            pl.semaphore_signal(
                barrier_sem, inc=1,
                device_id=(my_pos ^ m,), device_id_type=pl.DeviceIdType.MESH,
            )
        pl.semaphore_wait(barrier_sem, 3)

        q = jnp.dot(x_ref[...], wq_ref[...],
                    preferred_element_type=jnp.float32)
        rows = lax.broadcasted_iota(jnp.int32, (Sq, Skv), 0) // BLK
        cols = lax.broadcasted_iota(jnp.int32, (Sq, Skv), 1) // BLK
        mask = cols <= rows

        for b in range(B):
            for h in range(H_loc):
                bh = b * H_loc + h
                qbh = q[b * Sq:(b + 1) * Sq, h * Dh:(h + 1) * Dh]
                s = lax.dot_general(
                    qbh.astype(jnp.bfloat16), k_ref[bh],
                    (((1,), (1,)), ((), ())),
                    preferred_element_type=jnp.float32,
                ) * 0.125
                s = jnp.where(mask, s, -1e9)
                mx = jnp.max(s, axis=-1, keepdims=True)
                w = jnp.exp(s - mx)
                w = w / jnp.sum(w, axis=-1, keepdims=True)
                ctx = jnp.dot(w.astype(jnp.bfloat16), v_ref[bh],
                              preferred_element_type=jnp.float32)
                ctx_ref[b * Sq:(b + 1) * Sq, h * Dh:(h + 1) * Dh] = (
                    ctx.astype(jnp.bfloat16))

        out_ref[...] = jnp.dot(ctx_ref[...], wo_ref[...],
                               preferred_element_type=jnp.float32)

        a_prev = my_pos * 0
        for step, m in enumerate(RS_MASKS):
            partner = my_pos ^ m
            keep = my_pos & m
            a_new = a_prev + keep
            send_blk = a_prev + (m - keep)
            nrows = m * R_BLK
            off = RS_OFF[m]
            st_ref[off:off + nrows, :] = out_ref[
                pl.ds(send_blk * R_BLK, nrows), :].astype(jnp.bfloat16)
            rdma = pltpu.make_async_remote_copy(
                src_ref=st_ref.at[pl.ds(off, nrows)],
                dst_ref=rs_ref.at[pl.ds(off, nrows)],
                send_sem=send_sems.at[step],
                recv_sem=recv_sems.at[step],
                device_id=(partner,),
                device_id_type=pl.DeviceIdType.MESH,
            )
            rdma.start()
            rdma.wait()
            out_ref[pl.ds(a_new * R_BLK, nrows), :] = (
                out_ref[pl.ds(a_new * R_BLK, nrows), :]
                + rs_ref[off:off + nrows, :].astype(jnp.float32))
            a_prev = a_new

        ag_ref[pl.ds(my_pos * R_BLK, R_BLK), :] = out_ref[
            pl.ds(my_pos * R_BLK, R_BLK), :].astype(jnp.bfloat16)
        for step, m in enumerate(AG_MASKS):
            partner = my_pos ^ m
            g_start = my_pos & ((~(m - 1)) & (N_DEV - 1))
            nrows = m * R_BLK
            rdma = pltpu.make_async_remote_copy(
                src_ref=ag_ref.at[pl.ds(g_start * R_BLK, nrows)],
                dst_ref=ag_ref.at[pl.ds(g_start * R_BLK, nrows)],
                send_sem=send_sems.at[3 + step],
                recv_sem=recv_sems.at[3 + step],
                device_id=(partner,),
                device_id_type=pl.DeviceIdType.MESH,
            )
            rdma.start()
            rdma.wait()

        out_ref[...] = ag_ref[...].astype(jnp.float32)

    out = pl.pallas_call(
        body,
        out_shape=jax.ShapeDtypeStruct((R, Dm), jnp.float32),
        in_specs=[pl.BlockSpec(memory_space=pltpu.VMEM)] * 5,
        out_specs=pl.BlockSpec(memory_space=pltpu.VMEM),
        scratch_shapes=[
            pltpu.VMEM((7 * R_BLK, Dm), jnp.bfloat16),
            pltpu.VMEM((7 * R_BLK, Dm), jnp.bfloat16),
            pltpu.VMEM((R, Dm), jnp.bfloat16),
            pltpu.VMEM((R, H_loc * Dh), jnp.bfloat16),
            pltpu.SemaphoreType.DMA((6,)),
            pltpu.SemaphoreType.DMA((6,)),
        ],
        compiler_params=pltpu.CompilerParams(collective_id=0),
    )(xb, wq, k_loc, v_loc, wo)

    return out.reshape(B, Sq, Dm)


# device time: 22786 ns/iter; 2.7920x vs baseline; 1.4252x over previous
import jax
import jax.numpy as jnp
from jax import lax
from jax.experimental import pallas as pl
from jax.experimental.pallas import tpu as pltpu

N_DEV = 8
BLK = 64


def kernel(x, Wq, K_ext, V_ext, Wo):
    B, Sq, Dm = x.shape
    _, Skv, _, Dh = K_ext.shape
    H_loc = Wq.shape[1] // Dh
    BH = B * H_loc
    R = B * Sq
    R_BLK = R // N_DEV

    my = lax.axis_index("i")

    xb = x.reshape(R, Dm).astype(jnp.bfloat16)
    wq = Wq.astype(jnp.bfloat16)
    wo = Wo.astype(jnp.bfloat16)
    k_loc = lax.dynamic_slice_in_dim(K_ext, my * H_loc, H_loc, axis=2)
    v_loc = lax.dynamic_slice_in_dim(V_ext, my * H_loc, H_loc, axis=2)
    k_loc = k_loc.transpose(0, 2, 1, 3).reshape(BH, Skv, Dh).astype(jnp.bfloat16)
    v_loc = v_loc.transpose(0, 2, 1, 3).reshape(BH, Skv, Dh).astype(jnp.bfloat16)

    def body(x_ref, wq_ref, k_ref, v_ref, wo_ref, out_ref,
             st_ref, rs_ref, ag_ref, ctx_ref,
             rs_send_sems, rs_recv_sems, ag_send_sems, ag_recv_sems):
        my_pos = lax.axis_index("i")

        barrier_sem = pltpu.get_barrier_semaphore()
        for d in range(1, N_DEV):
            pl.semaphore_signal(
                barrier_sem, inc=1,
                device_id=((my_pos + d) % N_DEV,),
                device_id_type=pl.DeviceIdType.MESH,
            )
        pl.semaphore_wait(barrier_sem, N_DEV - 1)

        q = jnp.dot(x_ref[...], wq_ref[...],
                    preferred_element_type=jnp.float32)
        rows = lax.broadcasted_iota(jnp.int32, (Sq, Skv), 0) // BLK
        cols = lax.broadcasted_iota(jnp.int32, (Sq, Skv), 1) // BLK
        mask = cols <= rows

        for b in range(B):
            for h in range(H_loc):
                bh = b * H_loc + h
                qbh = q[b * Sq:(b + 1) * Sq, h * Dh:(h + 1) * Dh]
                s = lax.dot_general(
                    qbh.astype(jnp.bfloat16), k_ref[bh],
                    (((1,), (1,)), ((), ())),
                    preferred_element_type=jnp.float32,
                ) * 0.125
                s = jnp.where(mask, s, -1e9)
                mx = jnp.max(s, axis=-1, keepdims=True)
                w = jnp.exp(s - mx)
                w = w / jnp.sum(w, axis=-1, keepdims=True)
                ctx = jnp.dot(w.astype(jnp.bfloat16), v_ref[bh],
                              preferred_element_type=jnp.float32)
                ctx_ref[b * Sq:(b + 1) * Sq, h * Dh:(h + 1) * Dh] = (
                    ctx.astype(jnp.bfloat16))

        partial = jnp.dot(ctx_ref[...], wo_ref[...],
                          preferred_element_type=jnp.float32)
        out_ref[...] = partial
        st_ref[...] = partial.astype(jnp.bfloat16)

        for d in range(1, N_DEV):
            j = (my_pos + d) % N_DEV
            rdma = pltpu.make_async_remote_copy(
                src_ref=st_ref.at[pl.ds(j * R_BLK, R_BLK)],
                dst_ref=rs_ref.at[my_pos],
                send_sem=rs_send_sems.at[j],
                recv_sem=rs_recv_sems.at[my_pos],
                device_id=(j,),
                device_id_type=pl.DeviceIdType.MESH,
            )
            rdma.start()

        acc = out_ref[pl.ds(my_pos * R_BLK, R_BLK), :]
        for d in range(1, N_DEV):
            i = (my_pos + d) % N_DEV
            recv = pltpu.make_async_remote_copy(
                src_ref=st_ref.at[pl.ds(i * R_BLK, R_BLK)],
                dst_ref=rs_ref.at[i],
                send_sem=rs_send_sems.at[i],
                recv_sem=rs_recv_sems.at[i],
                device_id=(i,),
                device_id_type=pl.DeviceIdType.MESH,
            )
            recv.wait_recv()
            acc = acc + rs_ref[i].astype(jnp.float32)

        out_ref[pl.ds(my_pos * R_BLK, R_BLK), :] = acc
        ag_ref[pl.ds(my_pos * R_BLK, R_BLK), :] = acc.astype(jnp.bfloat16)
        for d in range(1, N_DEV):
            j = (my_pos + d) % N_DEV
            rdma = pltpu.make_async_remote_copy(
                src_ref=ag_ref.at[pl.ds(my_pos * R_BLK, R_BLK)],
                dst_ref=ag_ref.at[pl.ds(my_pos * R_BLK, R_BLK)],
                send_sem=ag_send_sems.at[j],
                recv_sem=ag_recv_sems.at[my_pos],
                device_id=(j,),
                device_id_type=pl.DeviceIdType.MESH,
            )
            rdma.start()

        for d in range(1, N_DEV):
            i = (my_pos + d) % N_DEV
            recv = pltpu.make_async_remote_copy(
                src_ref=ag_ref.at[pl.ds(i * R_BLK, R_BLK)],
                dst_ref=ag_ref.at[pl.ds(i * R_BLK, R_BLK)],
                send_sem=ag_send_sems.at[i],
                recv_sem=ag_recv_sems.at[i],
                device_id=(i,),
                device_id_type=pl.DeviceIdType.MESH,
            )
            recv.wait_recv()
            out_ref[pl.ds(i * R_BLK, R_BLK), :] = (
                ag_ref[pl.ds(i * R_BLK, R_BLK), :].astype(jnp.float32))

        for d in range(1, N_DEV):
            j = (my_pos + d) % N_DEV
            send = pltpu.make_async_remote_copy(
                src_ref=st_ref.at[pl.ds(j * R_BLK, R_BLK)],
                dst_ref=rs_ref.at[my_pos],
                send_sem=rs_send_sems.at[j],
                recv_sem=rs_recv_sems.at[my_pos],
                device_id=(j,),
                device_id_type=pl.DeviceIdType.MESH,
            )
            send.wait_send()
            send2 = pltpu.make_async_remote_copy(
                src_ref=ag_ref.at[pl.ds(my_pos * R_BLK, R_BLK)],
                dst_ref=ag_ref.at[pl.ds(my_pos * R_BLK, R_BLK)],
                send_sem=ag_send_sems.at[j],
                recv_sem=ag_recv_sems.at[my_pos],
                device_id=(j,),
                device_id_type=pl.DeviceIdType.MESH,
            )
            send2.wait_send()

    out = pl.pallas_call(
        body,
        out_shape=jax.ShapeDtypeStruct((R, Dm), jnp.float32),
        in_specs=[pl.BlockSpec(memory_space=pltpu.VMEM)] * 5,
        out_specs=pl.BlockSpec(memory_space=pltpu.VMEM),
        scratch_shapes=[
            pltpu.VMEM((R, Dm), jnp.bfloat16),
            pltpu.VMEM((N_DEV, R_BLK, Dm), jnp.bfloat16),
            pltpu.VMEM((R, Dm), jnp.bfloat16),
            pltpu.VMEM((R, H_loc * Dh), jnp.bfloat16),
            pltpu.SemaphoreType.DMA((N_DEV,)),
            pltpu.SemaphoreType.DMA((N_DEV,)),
            pltpu.SemaphoreType.DMA((N_DEV,)),
            pltpu.SemaphoreType.DMA((N_DEV,)),
        ],
        compiler_params=pltpu.CompilerParams(collective_id=0),
    )(xb, wq, k_loc, v_loc, wo)

    return out.reshape(B, Sq, Dm)
